# baseline (speedup 1.0000x reference)
import dataclasses
import functools

import jax
import jax.numpy as jnp
from jax import lax
from jax.experimental import pallas as pl
from jax.experimental.pallas import tpu as pltpu
from jax.experimental.pallas import tpu_sc as plsc

N_NODES = 10000
DEG = 32
D = 128
NC, NS = 2, 16
NW = NC * NS
N_PAD = 10240
NPW = N_PAD // NW
CH = 80
CHUNKS = NPW // CH
L = 16
NPAD_FLAT = (N_PAD - N_NODES) * DEG


def _sc_gather_sum(nbr_flat, pad_flat, emb):
    mesh = plsc.VectorSubcoreMesh(core_axis_name="c", subcore_axis_name="s")
    cp = pltpu.CompilerParams()
    if "needs_layout_passes" in pltpu.CompilerParams.__dataclass_fields__:
        cp = dataclasses.replace(cp, needs_layout_passes=False)

    @functools.partial(
        pl.kernel,
        out_type=jax.ShapeDtypeStruct((N_PAD, D), jnp.float32),
        mesh=mesh,
        compiler_params=cp,
        scratch_types=[
            pltpu.VMEM((NPW * DEG,), jnp.int32),
            pltpu.VMEM((DEG * CH,), jnp.int32),
            pltpu.VMEM((DEG * CH,), jnp.int32),
            pltpu.VMEM((CH, D), jnp.float32),
            pltpu.VMEM((CH, D), jnp.float32),
            pltpu.SemaphoreType.DMA,
            pltpu.SemaphoreType.DMA,
            pltpu.SemaphoreType.DMA,
            pltpu.SemaphoreType.DMA,
        ],
    )
    def k(nbr_hbm, pad_hbm, emb_hbm, out_hbm, raw_v, trans0, trans1,
          acc0, acc1, semA0, semA1, semO0, semO1):
        sid = lax.axis_index("s")
        wid = sid * NC + lax.axis_index("c")
        base = wid * NPW
        trans = (trans0, trans1)
        acc = (acc0, acc1)
        semA = (semA0, semA1)
        semO = (semO0, semO1)

        REAL_LAST = N_NODES * DEG - (NW - 1) * NPW * DEG

        @pl.when(wid < NW - 1)
        def _():
            pltpu.sync_copy(
                nbr_hbm.at[pl.ds(base * DEG, NPW * DEG)], raw_v
            )

        @pl.when(wid == NW - 1)
        def _():
            pltpu.sync_copy(
                nbr_hbm.at[pl.ds((NW - 1) * NPW * DEG, REAL_LAST)],
                raw_v.at[pl.ds(0, REAL_LAST)],
            )
            pltpu.sync_copy(
                pad_hbm,
                raw_v.at[pl.ds(REAL_LAST, NPAD_FLAT)],
            )

        iota_deg = lax.iota(jnp.int32, L) * DEG

        def transpose(c, p):
            t = trans[p]

            @pl.loop(0, DEG)
            def _(d):
                for g in range(CH // L):
                    src = iota_deg + ((c * CH + g * L) * DEG + d)
                    t[pl.ds(d * CH + g * L, L)] = plsc.load_gather(
                        raw_v, [src]
                    )

        def zero(p):
            a = acc[p]

            @pl.loop(0, CH)
            def _(i):
                @pl.loop(0, D, step=L)
                def _(col):
                    a[i, pl.ds(col, L)] = jnp.zeros((L,), jnp.float32)

        def fire_adds(c, p):
            for d in range(DEG):
                pltpu.async_copy(
                    emb_hbm.at[trans[p].at[pl.ds(d * CH, CH)]],
                    acc[p],
                    semA[p],
                    add=True,
                )

        def drain_adds(c, p):
            for d in range(DEG):
                pltpu.make_async_copy(
                    emb_hbm.at[trans[p].at[pl.ds(d * CH, CH)]],
                    acc[p],
                    semA[p],
                ).wait()

        def fire_out(c, p):
            pltpu.async_copy(
                acc[p], out_hbm.at[pl.ds(base + c * CH, CH)], semO[p]
            )

        def drain_out(c, p):
            pltpu.make_async_copy(
                acc[p], out_hbm.at[pl.ds(base + c * CH, CH)], semO[p]
            ).wait()

        transpose(0, 0)
        zero(0)
        zero(1)
        fire_adds(0, 0)
        for c in range(CHUNKS):
            p = c % 2
            if c + 1 < CHUNKS:
                if c >= 1:
                    drain_out(c - 1, 1 - p)
                    zero(1 - p)
                transpose(c + 1, 1 - p)
                fire_adds(c + 1, 1 - p)
            drain_adds(c, p)
            fire_out(c, p)
        drain_out(CHUNKS - 2, CHUNKS % 2)
        drain_out(CHUNKS - 1, (CHUNKS - 1) % 2)

    return k(nbr_flat, pad_flat, emb)


def _tc_post(sums, W):
    BLK = 2000

    def body(x_ref, w_ref, o_ref):
        y = lax.dot_general(
            x_ref[...], w_ref[...],
            (((1,), (1,)), ((), ())),
            preferred_element_type=jnp.float32,
        )
        y = jnp.maximum(y, 0.0)
        norm = jnp.sqrt(jnp.sum(y * y, axis=1, keepdims=True))
        o_ref[...] = y / jnp.maximum(norm, 1e-12)

    return pl.pallas_call(
        body,
        grid=(N_NODES // BLK,),
        in_specs=[
            pl.BlockSpec((BLK, D), lambda i: (i, 0)),
            pl.BlockSpec((D, D), lambda i: (0, 0)),
        ],
        out_specs=pl.BlockSpec((BLK, D), lambda i: (i, 0)),
        out_shape=jax.ShapeDtypeStruct((N_NODES, D), jnp.float32),
    )(sums, W)


def kernel(neighbors, emb_features, W):
    pad_flat = jnp.arange(NPAD_FLAT, dtype=jnp.int32) % N_NODES
    sums = _sc_gather_sum(neighbors.reshape(-1), pad_flat, emb_features)
    return _tc_post(sums, W)


# speedup vs baseline: 1.0042x; 1.0042x over previous
import dataclasses
import functools

import jax
import jax.numpy as jnp
from jax import lax
from jax.experimental import pallas as pl
from jax.experimental.pallas import tpu as pltpu
from jax.experimental.pallas import tpu_sc as plsc

N_NODES = 10000
DEG = 32
D = 128
NC, NS = 2, 16
NW = NC * NS
N_PAD = 10240
NPW = N_PAD // NW
CH = 80
CHUNKS = NPW // CH
L = 16
NPAD_FLAT = (N_PAD - N_NODES) * DEG


def _sc_gather_sum(nbr_flat, pad_flat, emb):
    mesh = plsc.VectorSubcoreMesh(core_axis_name="c", subcore_axis_name="s")
    cp = pltpu.CompilerParams()
    if "needs_layout_passes" in pltpu.CompilerParams.__dataclass_fields__:
        cp = dataclasses.replace(cp, needs_layout_passes=False)

    @functools.partial(
        pl.kernel,
        out_type=jax.ShapeDtypeStruct((N_PAD, D), jnp.float32),
        mesh=mesh,
        compiler_params=cp,
        scratch_types=[
            pltpu.VMEM((NPW * DEG,), jnp.int32),
            pltpu.VMEM((DEG * CH,), jnp.int32),
            pltpu.VMEM((DEG * CH,), jnp.int32),
            pltpu.VMEM((CH, D), jnp.float32),
            pltpu.VMEM((CH, D), jnp.float32),
            pltpu.SemaphoreType.DMA,
            pltpu.SemaphoreType.DMA,
            pltpu.SemaphoreType.DMA,
            pltpu.SemaphoreType.DMA,
        ],
    )
    def k(nbr_hbm, pad_hbm, emb_hbm, out_hbm, raw_v, trans0, trans1,
          acc0, acc1, semA0, semA1, semO0, semO1):
        sid = lax.axis_index("s")
        wid = sid * NC + lax.axis_index("c")
        base = wid * NPW
        trans = (trans0, trans1)
        acc = (acc0, acc1)
        semA = (semA0, semA1)
        semO = (semO0, semO1)

        REAL_LAST = N_NODES * DEG - (NW - 1) * NPW * DEG

        @pl.when(wid < NW - 1)
        def _():
            pltpu.sync_copy(
                nbr_hbm.at[pl.ds(base * DEG, NPW * DEG)], raw_v
            )

        @pl.when(wid == NW - 1)
        def _():
            pltpu.sync_copy(
                nbr_hbm.at[pl.ds((NW - 1) * NPW * DEG, REAL_LAST)],
                raw_v.at[pl.ds(0, REAL_LAST)],
            )
            pltpu.sync_copy(
                pad_hbm,
                raw_v.at[pl.ds(REAL_LAST, NPAD_FLAT)],
            )

        iota_deg = lax.iota(jnp.int32, L) * DEG

        def trans_fire(c, p):
            t = trans[p]
            for d in range(DEG):
                for g in range(CH // L):
                    src = iota_deg + ((c * CH + g * L) * DEG + d)
                    t[pl.ds(d * CH + g * L, L)] = plsc.load_gather(
                        raw_v, [src]
                    )
                pltpu.async_copy(
                    emb_hbm.at[t.at[pl.ds(d * CH, CH)]],
                    acc[p],
                    semA[p],
                    add=True,
                )

        def zero(p):
            a = acc[p]

            @pl.loop(0, CH)
            def _(i):
                @pl.loop(0, D, step=L)
                def _(col):
                    a[i, pl.ds(col, L)] = jnp.zeros((L,), jnp.float32)

        def drain_adds(c, p):
            for d in range(DEG):
                pltpu.make_async_copy(
                    emb_hbm.at[trans[p].at[pl.ds(d * CH, CH)]],
                    acc[p],
                    semA[p],
                ).wait()

        def fire_out(c, p):
            pltpu.async_copy(
                acc[p], out_hbm.at[pl.ds(base + c * CH, CH)], semO[p]
            )

        def drain_out(c, p):
            pltpu.make_async_copy(
                acc[p], out_hbm.at[pl.ds(base + c * CH, CH)], semO[p]
            ).wait()

        zero(0)
        zero(1)
        trans_fire(0, 0)
        for c in range(CHUNKS):
            p = c % 2
            if c + 1 < CHUNKS:
                if c >= 1:
                    drain_out(c - 1, 1 - p)
                    zero(1 - p)
                trans_fire(c + 1, 1 - p)
            drain_adds(c, p)
            fire_out(c, p)
        drain_out(CHUNKS - 2, CHUNKS % 2)
        drain_out(CHUNKS - 1, (CHUNKS - 1) % 2)

    return k(nbr_flat, pad_flat, emb)


def _tc_post(sums, W):
    BLK = 2000

    def body(x_ref, w_ref, o_ref):
        y = lax.dot_general(
            x_ref[...], w_ref[...],
            (((1,), (1,)), ((), ())),
            preferred_element_type=jnp.float32,
        )
        y = jnp.maximum(y, 0.0)
        norm = jnp.sqrt(jnp.sum(y * y, axis=1, keepdims=True))
        o_ref[...] = y / jnp.maximum(norm, 1e-12)

    return pl.pallas_call(
        body,
        grid=(N_NODES // BLK,),
        in_specs=[
            pl.BlockSpec((BLK, D), lambda i: (i, 0)),
            pl.BlockSpec((D, D), lambda i: (0, 0)),
        ],
        out_specs=pl.BlockSpec((BLK, D), lambda i: (i, 0)),
        out_shape=jax.ShapeDtypeStruct((N_NODES, D), jnp.float32),
    )(sums, W)


def kernel(neighbors, emb_features, W):
    pad_flat = jnp.arange(NPAD_FLAT, dtype=jnp.int32) % N_NODES
    sums = _sc_gather_sum(neighbors.reshape(-1), pad_flat, emb_features)
    return _tc_post(sums, W)


# speedup vs baseline: 1.0456x; 1.0412x over previous
import functools

import jax
import jax.numpy as jnp
from jax import lax
from jax.experimental import pallas as pl
from jax.experimental.pallas import tpu as pltpu
from jax.experimental.pallas import tpu_sc as plsc

N_NODES = 10000
DEG = 32
D = 128
NC, NS = 2, 16
NW = NC * NS
N_PAD = 10240
NPW = N_PAD // NW
CH = 80
CHUNKS = NPW // CH


def _sc_gather_sum(nbr_t, emb):
    mesh = plsc.VectorSubcoreMesh(core_axis_name="c", subcore_axis_name="s")

    @functools.partial(
        pl.kernel,
        out_type=jax.ShapeDtypeStruct((N_PAD, D), jnp.float32),
        mesh=mesh,
        scratch_types=[
            pltpu.VMEM((DEG * NPW,), jnp.int32),
            pltpu.VMEM((CH, D), jnp.float32),
            pltpu.VMEM((CH, D), jnp.float32),
            pltpu.SemaphoreType.DMA,
            pltpu.SemaphoreType.DMA,
            pltpu.SemaphoreType.DMA,
            pltpu.SemaphoreType.DMA,
        ],
    )
    def k(nbr_hbm, emb_hbm, out_hbm, idx_all, acc0, acc1, semA0, semA1,
          semO0, semO1):
        sid = lax.axis_index("s")
        wid = sid * NC + lax.axis_index("c")
        base = wid * NPW
        acc = (acc0, acc1)
        semA = (semA0, semA1)
        semO = (semO0, semO1)

        idx_cp = pltpu.async_copy(
            nbr_hbm.at[pl.ds(wid * DEG * NPW, DEG * NPW)], idx_all, semO0
        )

        def zero(p):
            a = acc[p]

            @pl.loop(0, CH)
            def _(i):
                @pl.loop(0, D, step=16)
                def _(col):
                    a[i, pl.ds(col, 16)] = jnp.zeros((16,), jnp.float32)

        def fire_adds(c, p):
            for d in range(DEG):
                pltpu.async_copy(
                    emb_hbm.at[idx_all.at[pl.ds(d * NPW + c * CH, CH)]],
                    acc[p],
                    semA[p],
                    add=True,
                )

        def drain_adds(c, p):
            for d in range(DEG):
                pltpu.make_async_copy(
                    emb_hbm.at[idx_all.at[pl.ds(d * NPW + c * CH, CH)]],
                    acc[p],
                    semA[p],
                ).wait()

        def fire_out(c, p):
            pltpu.async_copy(
                acc[p], out_hbm.at[pl.ds(base + c * CH, CH)], semO[p]
            )

        def drain_out(c, p):
            pltpu.make_async_copy(
                acc[p], out_hbm.at[pl.ds(base + c * CH, CH)], semO[p]
            ).wait()

        zero(0)
        zero(1)
        idx_cp.wait()
        fire_adds(0, 0)
        for c in range(CHUNKS):
            p = c % 2
            if c + 1 < CHUNKS:
                if c >= 1:
                    drain_out(c - 1, 1 - p)
                    zero(1 - p)
                fire_adds(c + 1, 1 - p)
            drain_adds(c, p)
            fire_out(c, p)
        drain_out(CHUNKS - 2, CHUNKS % 2)
        drain_out(CHUNKS - 1, (CHUNKS - 1) % 2)

    return k(nbr_t, emb)


def _tc_post(sums, W):
    BLK = 2000

    def body(x_ref, w_ref, o_ref):
        y = lax.dot_general(
            x_ref[...], w_ref[...],
            (((1,), (1,)), ((), ())),
            preferred_element_type=jnp.float32,
        )
        y = jnp.maximum(y, 0.0)
        norm = jnp.sqrt(jnp.sum(y * y, axis=1, keepdims=True))
        o_ref[...] = y / jnp.maximum(norm, 1e-12)

    return pl.pallas_call(
        body,
        grid=(N_NODES // BLK,),
        in_specs=[
            pl.BlockSpec((BLK, D), lambda i: (i, 0)),
            pl.BlockSpec((D, D), lambda i: (0, 0)),
        ],
        out_specs=pl.BlockSpec((BLK, D), lambda i: (i, 0)),
        out_shape=jax.ShapeDtypeStruct((N_NODES, D), jnp.float32),
    )(sums, W)


def kernel(neighbors, emb_features, W):
    pad_idx = (
        jnp.arange((N_PAD - N_NODES) * DEG, dtype=jnp.int32) % N_NODES
    ).reshape(N_PAD - N_NODES, DEG)
    nbr = jnp.concatenate([neighbors, pad_idx], axis=0)
    nbr_t = (
        nbr.reshape(NW, NPW, DEG).transpose(0, 2, 1).reshape(-1)
    )
    sums = _sc_gather_sum(nbr_t, emb_features)
    return _tc_post(sums, W)
